# Initial kernel scaffold; baseline (speedup 1.0000x reference)
import jax
import jax.numpy as jnp
from jax.experimental import pallas as pl


def kernel(x, edge_index, edge_weight, batch, W1, b1, W2, b2, W3, b3, Wl, bl):
    raise NotImplementedError("write your pallas kernel here")


# speedup vs baseline: 3.6911x; 3.6911x over previous
import functools

import jax
import jax.numpy as jnp
from jax import lax
from jax.experimental import pallas as pl
from jax.experimental.pallas import tpu as pltpu
from jax.experimental.pallas import tpu_sc as plsc

N_NODES = 10000
NP = 10240
N_EDGES = 320000
EP = 327680
NBC = 640
BPT = NBC // 16
RPT = NP // 16
N_GRAPHS = 16

@functools.cache
def _make_agg(fc, npass=1):
    mesh = plsc.VectorSubcoreMesh(
        core_axis_name="c", subcore_axis_name="s", num_cores=2, num_subcores=16
    )
    nblk = 2 * npass

    def body(h_hbm, src_hbm, dst_hbm, ew_hbm, out_hbm,
             sidx, didx, ew_v, rows, acc, sem):
        c = lax.axis_index("c")
        s = lax.axis_index("s")

        for p in range(npass):
            blk = c * npass + p
            row_off = blk * NP

            def zrow(r, carry):
                for j in range(fc // 16):
                    rows[r, pl.ds(j * 16, 16)] = jnp.zeros((16,), jnp.float32)
                return carry
            lax.fori_loop(0, 512, zrow, 0)
            pltpu.sync_copy(rows, acc.at[pl.ds(s * RPT, 512)])
            pltpu.sync_copy(
                rows.at[pl.ds(0, 128)], acc.at[pl.ds(s * RPT + 512, 128)]
            )
            plsc.subcore_barrier()

            def chunk(bi, carry):
                pltpu.sync_copy(src_hbm.at[bi], sidx)
                pltpu.sync_copy(dst_hbm.at[bi], didx)
                pltpu.sync_copy(ew_hbm.at[bi], ew_v)

                for j in range(4):
                    for k in range(8):
                        sl = (j, pl.ds(k * 16, 16))
                        sidx[sl] = sidx[sl] + row_off
                cps = [
                    pltpu.async_copy(
                        h_hbm.at[sidx.at[j]], rows.at[pl.ds(j * 128, 128)], sem
                    )
                    for j in range(4)
                ]
                for cp in cps:
                    cp.wait()

                def grp(g, carry2):
                    base = g * 16
                    w16 = ew_v[pl.ds(base, 16)]
                    for i in range(16):
                        wv = jnp.full((16,), w16[i], jnp.float32)
                        e = base + i
                        for j in range(fc // 16):
                            sl = (e, pl.ds(j * 16, 16))
                            rows[sl] = rows[sl] * wv
                    return carry2
                lax.fori_loop(0, 32, grp, 0)

                for j in range(4):
                    pltpu.sync_copy(
                        rows.at[pl.ds(j * 128, 128)], acc.at[didx.at[j]],
                        add=True,
                    )
                return carry
            lax.fori_loop(s * BPT, (s + 1) * BPT, chunk, 0)

            plsc.subcore_barrier()
            pltpu.sync_copy(
                acc.at[pl.ds(s * RPT, RPT)],
                out_hbm.at[pl.ds(row_off + s * RPT, RPT)],
            )

    return pl.kernel(
        body,
        out_type=jax.ShapeDtypeStruct((nblk * NP, fc), jnp.float32),
        mesh=mesh,
        compiler_params=pltpu.CompilerParams(use_tc_tiling_on_sc=False),
        scratch_types=[
            pltpu.VMEM((4, 128), jnp.int32),
            pltpu.VMEM((4, 128), jnp.int32),
            pltpu.VMEM((512,), jnp.float32),
            pltpu.VMEM((512, fc), jnp.float32),
            pltpu.VMEM_SHARED((NP, fc), jnp.float32),
            pltpu.SemaphoreType.DMA,
        ],
    )


def _tc1_body(x_ref, w1_ref, out_ref):
    h = jnp.dot(x_ref[...], w1_ref[...], preferred_element_type=jnp.float32)
    out_ref[...] = jnp.concatenate([h[:, :32], h[:, 32:]], axis=0)


def _tc2_body(agg_ref, b1_ref, w2_ref, x1_ref, out_ref):
    agg = agg_ref[...]
    x1 = jnp.maximum(
        jnp.concatenate([agg[:NP], agg[NP:]], axis=1) + b1_ref[...], 0.0
    )
    x1_ref[...] = x1
    h2 = jnp.dot(x1, w2_ref[...], preferred_element_type=jnp.float32)
    out_ref[...] = jnp.concatenate([h2[:, :64], h2[:, 64:]], axis=0)


def _tc3_body(agg_ref, b2_ref, x1_ref, w3_ref, out_ref):
    agg = agg_ref[...]
    x2 = jnp.maximum(
        jnp.concatenate([agg[:NP], agg[NP:]], axis=1) + b2_ref[...], 0.0
    )
    xc = jnp.concatenate([x1_ref[...], x2], axis=1)
    h3 = jnp.dot(xc, w3_ref[...], preferred_element_type=jnp.float32)
    out_ref[...] = jnp.concatenate(
        [h3[:, 64 * b:64 * (b + 1)] for b in range(4)], axis=0
    )


def _tc4_body(agg_ref, b3_ref, batch_ref, wl_ref, bl_ref, out_ref):
    agg = agg_ref[...]
    x3 = jnp.maximum(
        jnp.concatenate([agg[NP * b:NP * (b + 1)] for b in range(4)], axis=1)
        + b3_ref[...],
        0.0,
    )
    bvec = batch_ref[...]
    pooled = jnp.concatenate(
        [
            jnp.max(jnp.where(bvec == g, x3, -jnp.inf), axis=0, keepdims=True)
            for g in range(N_GRAPHS)
        ],
        axis=0,
    )
    logits = (
        jnp.dot(pooled, wl_ref[...], preferred_element_type=jnp.float32)
        + bl_ref[...]
    )
    mx = jnp.max(logits, axis=1, keepdims=True)
    lse = jnp.log(jnp.sum(jnp.exp(logits - mx), axis=1, keepdims=True)) + mx
    out_ref[...] = logits - lse


_tc1 = pl.pallas_call(
    _tc1_body, out_shape=jax.ShapeDtypeStruct((2 * NP, 32), jnp.float32)
)
_tc2 = pl.pallas_call(
    _tc2_body,
    out_shape=[
        jax.ShapeDtypeStruct((NP, 64), jnp.float32),
        jax.ShapeDtypeStruct((2 * NP, 64), jnp.float32),
    ],
)
_tc3 = pl.pallas_call(
    _tc3_body, out_shape=jax.ShapeDtypeStruct((4 * NP, 64), jnp.float32)
)
_tc4 = pl.pallas_call(
    _tc4_body, out_shape=jax.ShapeDtypeStruct((N_GRAPHS, 40), jnp.float32)
)


def kernel(x, edge_index, edge_weight, batch, W1, b1, W2, b2, W3, b3, Wl, bl):
    x = x.astype(jnp.float32)
    src = edge_index[0].astype(jnp.int32)
    dst = edge_index[1].astype(jnp.int32)
    ew = edge_weight.astype(jnp.float32)

    npad = EP - N_EDGES
    pad_idx = (jnp.arange(npad, dtype=jnp.int32) * 97) % N_NODES
    srcp = jnp.concatenate([src, pad_idx]).reshape(NBC, 4, 128)
    dstp = jnp.concatenate([dst, pad_idx]).reshape(NBC, 4, 128)
    ewp = jnp.concatenate([ew, jnp.zeros((npad,), jnp.float32)]).reshape(NBC, 512)

    xp = jnp.pad(x, ((0, NP - N_NODES), (0, 0)))
    batch_p = jnp.concatenate(
        [batch.astype(jnp.int32), jnp.full((NP - N_NODES,), N_GRAPHS, jnp.int32)]
    ).reshape(NP, 1)
    W3p = jnp.pad(W3, ((0, 0), (0, 2)))
    b3p = jnp.pad(b3, (0, 2))
    Wlp = jnp.pad(Wl, ((0, 2), (0, 0)))

    h1 = _tc1(xp, W1)
    agg1 = _make_agg(32)(h1, srcp, dstp, ewp)
    x1, h2 = _tc2(agg1, b1, W2)
    agg2 = _make_agg(64)(h2, srcp, dstp, ewp)
    h3 = _tc3(agg2, b2, x1, W3p)
    agg3 = _make_agg(64, 2)(h3, srcp, dstp, ewp)
    return _tc4(agg3, b3p, batch_p, Wlp, bl)


# speedup vs baseline: 4.2185x; 1.1429x over previous
import functools

import jax
import jax.numpy as jnp
from jax import lax
from jax.experimental import pallas as pl
from jax.experimental.pallas import tpu as pltpu
from jax.experimental.pallas import tpu_sc as plsc

N_NODES = 10000
NP = 10240
N_EDGES = 320000
EP = 327680
NBC = 640
BPT = NBC // 16
RPT = NP // 16
N_GRAPHS = 16

@functools.cache
def _make_agg(fc, npass=1):
    mesh = plsc.VectorSubcoreMesh(
        core_axis_name="c", subcore_axis_name="s", num_cores=2, num_subcores=16
    )
    nblk = 2 * npass

    def body(h_hbm, src_hbm, dst_hbm, ew_hbm, out_hbm,
             sidx, didx, ew_v, rows, acc, sem):
        c = lax.axis_index("c")
        s = lax.axis_index("s")

        for p in range(npass):
            blk = c * npass + p
            row_off = blk * NP

            def zrow(r, carry):
                for j in range(fc // 16):
                    rows[r, pl.ds(j * 16, 16)] = jnp.zeros((16,), jnp.float32)
                return carry
            lax.fori_loop(0, 512, zrow, 0)
            pltpu.sync_copy(rows, acc.at[pl.ds(s * RPT, 512)])
            pltpu.sync_copy(
                rows.at[pl.ds(0, 128)], acc.at[pl.ds(s * RPT + 512, 128)]
            )
            plsc.subcore_barrier()

            def chunk(bi, carry):
                pltpu.sync_copy(src_hbm.at[bi], sidx)
                pltpu.sync_copy(dst_hbm.at[bi], didx)
                pltpu.sync_copy(ew_hbm.at[bi], ew_v)

                for j in range(4):
                    for k in range(8):
                        sl = (j, pl.ds(k * 16, 16))
                        sidx[sl] = sidx[sl] + row_off
                cps = [
                    pltpu.async_copy(
                        h_hbm.at[sidx.at[j]], rows.at[pl.ds(j * 128, 128)],
                        sem.at[j],
                    )
                    for j in range(4)
                ]
                for j in range(4):
                    cps[j].wait()

                    def grp(g, carry2, j=j):
                        base = j * 128 + g * 16
                        w16 = ew_v[pl.ds(base, 16)]
                        for i in range(16):
                            wv = jnp.full((16,), w16[i], jnp.float32)
                            e = base + i
                            for jj in range(fc // 16):
                                sl = (e, pl.ds(jj * 16, 16))
                                rows[sl] = rows[sl] * wv
                        return carry2
                    lax.fori_loop(0, 8, grp, 0)

                    pltpu.sync_copy(
                        rows.at[pl.ds(j * 128, 128)], acc.at[didx.at[j]],
                        add=True,
                    )
                return carry
            lax.fori_loop(s * BPT, (s + 1) * BPT, chunk, 0)

            plsc.subcore_barrier()
            pltpu.sync_copy(
                acc.at[pl.ds(s * RPT, RPT)],
                out_hbm.at[pl.ds(row_off + s * RPT, RPT)],
            )

    return pl.kernel(
        body,
        out_type=jax.ShapeDtypeStruct((nblk * NP, fc), jnp.float32),
        mesh=mesh,
        compiler_params=pltpu.CompilerParams(use_tc_tiling_on_sc=False),
        scratch_types=[
            pltpu.VMEM((4, 128), jnp.int32),
            pltpu.VMEM((4, 128), jnp.int32),
            pltpu.VMEM((512,), jnp.float32),
            pltpu.VMEM((512, fc), jnp.float32),
            pltpu.VMEM_SHARED((NP, fc), jnp.float32),
            pltpu.SemaphoreType.DMA((4,)),
        ],
    )


def _tc1_body(x_ref, w1_ref, out_ref):
    h = jnp.dot(x_ref[...], w1_ref[...], preferred_element_type=jnp.float32)
    out_ref[...] = jnp.concatenate([h[:, :32], h[:, 32:]], axis=0)


def _tc2_body(agg_ref, b1_ref, w2_ref, x1_ref, out_ref):
    agg = agg_ref[...]
    x1 = jnp.maximum(
        jnp.concatenate([agg[:NP], agg[NP:]], axis=1) + b1_ref[...], 0.0
    )
    x1_ref[...] = x1
    h2 = jnp.dot(x1, w2_ref[...], preferred_element_type=jnp.float32)
    out_ref[...] = jnp.concatenate([h2[:, :64], h2[:, 64:]], axis=0)


def _tc3_body(agg_ref, b2_ref, x1_ref, w3_ref, out_ref):
    agg = agg_ref[...]
    x2 = jnp.maximum(
        jnp.concatenate([agg[:NP], agg[NP:]], axis=1) + b2_ref[...], 0.0
    )
    xc = jnp.concatenate([x1_ref[...], x2], axis=1)
    h3 = jnp.dot(xc, w3_ref[...], preferred_element_type=jnp.float32)
    out_ref[...] = jnp.concatenate(
        [h3[:, 64 * b:64 * (b + 1)] for b in range(4)], axis=0
    )


def _tc4_body(agg_ref, b3_ref, batch_ref, wl_ref, bl_ref, out_ref):
    agg = agg_ref[...]
    x3 = jnp.maximum(
        jnp.concatenate([agg[NP * b:NP * (b + 1)] for b in range(4)], axis=1)
        + b3_ref[...],
        0.0,
    )
    bvec = batch_ref[...]
    pooled = jnp.concatenate(
        [
            jnp.max(jnp.where(bvec == g, x3, -jnp.inf), axis=0, keepdims=True)
            for g in range(N_GRAPHS)
        ],
        axis=0,
    )
    logits = (
        jnp.dot(pooled, wl_ref[...], preferred_element_type=jnp.float32)
        + bl_ref[...]
    )
    mx = jnp.max(logits, axis=1, keepdims=True)
    lse = jnp.log(jnp.sum(jnp.exp(logits - mx), axis=1, keepdims=True)) + mx
    out_ref[...] = logits - lse


_tc1 = pl.pallas_call(
    _tc1_body, out_shape=jax.ShapeDtypeStruct((2 * NP, 32), jnp.float32)
)
_tc2 = pl.pallas_call(
    _tc2_body,
    out_shape=[
        jax.ShapeDtypeStruct((NP, 64), jnp.float32),
        jax.ShapeDtypeStruct((2 * NP, 64), jnp.float32),
    ],
)
_tc3 = pl.pallas_call(
    _tc3_body, out_shape=jax.ShapeDtypeStruct((4 * NP, 64), jnp.float32)
)
_tc4 = pl.pallas_call(
    _tc4_body, out_shape=jax.ShapeDtypeStruct((N_GRAPHS, 40), jnp.float32)
)


def kernel(x, edge_index, edge_weight, batch, W1, b1, W2, b2, W3, b3, Wl, bl):
    x = x.astype(jnp.float32)
    src = edge_index[0].astype(jnp.int32)
    dst = edge_index[1].astype(jnp.int32)
    ew = edge_weight.astype(jnp.float32)

    npad = EP - N_EDGES
    pad_idx = (jnp.arange(npad, dtype=jnp.int32) * 97) % N_NODES
    srcp = jnp.concatenate([src, pad_idx]).reshape(NBC, 4, 128)
    dstp = jnp.concatenate([dst, pad_idx]).reshape(NBC, 4, 128)
    ewp = jnp.concatenate([ew, jnp.zeros((npad,), jnp.float32)]).reshape(NBC, 512)

    xp = jnp.pad(x, ((0, NP - N_NODES), (0, 0)))
    batch_p = jnp.concatenate(
        [batch.astype(jnp.int32), jnp.full((NP - N_NODES,), N_GRAPHS, jnp.int32)]
    ).reshape(NP, 1)
    W3p = jnp.pad(W3, ((0, 0), (0, 2)))
    b3p = jnp.pad(b3, (0, 2))
    Wlp = jnp.pad(Wl, ((0, 2), (0, 0)))

    h1 = _tc1(xp, W1)
    agg1 = _make_agg(32)(h1, srcp, dstp, ewp)
    x1, h2 = _tc2(agg1, b1, W2)
    agg2 = _make_agg(64)(h2, srcp, dstp, ewp)
    h3 = _tc3(agg2, b2, x1, W3p)
    agg3 = _make_agg(64, 2)(h3, srcp, dstp, ewp)
    return _tc4(agg3, b3p, batch_p, Wlp, bl)
